# Initial kernel scaffold; baseline (speedup 1.0000x reference)
import jax
import jax.numpy as jnp
from jax.experimental import pallas as pl


def kernel(x, edge_index, W, b):
    raise NotImplementedError("write your pallas kernel here")


# speedup vs baseline: 10.8295x; 10.8295x over previous
import functools

import jax
import jax.numpy as jnp
from jax import lax
from jax.experimental import pallas as pl
from jax.experimental.pallas import tpu as pltpu
from jax.experimental.pallas import tpu_sc as plsc

N = 10000
E = 320000
D = 128

NC = 2
NS = 16
NW = NC * NS
CH = 128
CPT = (E + NW * CH - 1) // (NW * CH)
EPAD = NW * CPT * CH
NPAD = 10112
STRIPE = NPAD // NS
NDEG = 10240
DSTRIPE = NDEG // NS

_mesh = plsc.VectorSubcoreMesh(
    core_axis_name="c", subcore_axis_name="s", num_cores=NC, num_subcores=NS
)


def _sc_body(x_hbm, src_hbm, dst_hbm, part_hbm, degp_hbm,
             acc_sh, deg_sh, src_v, dst_v, ones_v, zbuf_v, rows_v, sem):
    c = lax.axis_index("c")
    s = lax.axis_index("s")
    wid = c * NS + s

    pltpu.sync_copy(src_hbm.at[wid], src_v)
    pltpu.sync_copy(dst_hbm.at[wid], dst_v)

    for k in range(CH // 16):
        ones_v[pl.ds(k * 16, 16)] = jnp.ones((16,), jnp.float32)

    def _zero_row(i, _):
        for k in range(D // 16):
            rows_v[i, pl.ds(k * 16, 16)] = jnp.zeros((16,), jnp.float32)
        return 0
    lax.fori_loop(0, CH, _zero_row, 0)
    for k in range(DSTRIPE // 16):
        zbuf_v[pl.ds(k * 16, 16)] = jnp.zeros((16,), jnp.float32)

    row0 = s * STRIPE
    nfull = STRIPE // CH
    rem = STRIPE - nfull * CH
    for k in range(nfull):
        pltpu.sync_copy(rows_v, acc_sh.at[pl.ds(row0 + k * CH, CH)])
    pltpu.sync_copy(rows_v.at[pl.ds(0, rem)],
                    acc_sh.at[pl.ds(row0 + nfull * CH, rem)])
    pltpu.sync_copy(zbuf_v, deg_sh.at[pl.ds(s * DSTRIPE, DSTRIPE)])

    plsc.subcore_barrier()

    def _chunk(j, _):
        pltpu.sync_copy(x_hbm.at[src_v.at[j]], rows_v)
        pltpu.sync_copy(rows_v, acc_sh.at[dst_v.at[j]], add=True)
        pltpu.sync_copy(ones_v, deg_sh.at[dst_v.at[j]], add=True)
        return 0
    lax.fori_loop(0, CPT, _chunk, 0)

    plsc.subcore_barrier()

    pltpu.sync_copy(acc_sh.at[pl.ds(row0, STRIPE)],
                    part_hbm.at[c, pl.ds(row0, STRIPE)])
    pltpu.sync_copy(deg_sh.at[pl.ds(s * DSTRIPE, DSTRIPE)],
                    degp_hbm.at[c, pl.ds(s * DSTRIPE, DSTRIPE)])


_sc_call = pl.kernel(
    _sc_body,
    out_type=(
        jax.ShapeDtypeStruct((NC, NPAD, D), jnp.float32),
        jax.ShapeDtypeStruct((NC, NDEG), jnp.float32),
    ),
    mesh=_mesh,
    scratch_types=[
        pltpu.VMEM_SHARED((NPAD, D), jnp.float32),
        pltpu.VMEM_SHARED((NDEG,), jnp.float32),
        pltpu.VMEM((CPT, CH), jnp.int32),
        pltpu.VMEM((CPT, CH), jnp.int32),
        pltpu.VMEM((CH,), jnp.float32),
        pltpu.VMEM((DSTRIPE,), jnp.float32),
        pltpu.VMEM((CH, D), jnp.float32),
        pltpu.SemaphoreType.DMA,
    ],
)


def _tc_body(part_ref, deg_ref, w_ref, b_ref, out_ref):
    p = part_ref[0] + part_ref[1]
    h = jnp.dot(p, w_ref[...], preferred_element_type=jnp.float32)
    deg = jnp.maximum(deg_ref[0] + deg_ref[1], 1.0)
    t = h / deg + b_ref[...]
    out_ref[...] = jnp.where(t >= 0.0, t, 0.01 * t)


_tc_call = pl.pallas_call(
    _tc_body,
    out_shape=jax.ShapeDtypeStruct((N, D), jnp.float32),
)


def kernel(x, edge_index, W, b):
    src = edge_index[0]
    dst = edge_index[1]
    pad = EPAD - E
    srcp = jnp.concatenate([src, jnp.zeros((pad,), jnp.int32)]).reshape(NW, CPT, CH)
    dstp = jnp.concatenate([dst, jnp.full((pad,), N, jnp.int32)]).reshape(NW, CPT, CH)
    part, degp = _sc_call(x, srcp, dstp)
    return _tc_call(part[:, :N, :], degp[:, :N, None], W, b.reshape(1, D))
